# baseline (speedup 1.0000x reference)
import functools

import jax
import jax.numpy as jnp
from jax import lax
from jax.experimental import pallas as pl
from jax.experimental.pallas import tpu as pltpu
from jax.experimental.pallas import tpu_sc as plsc

_NN = 10000
_NE = 160000
_D = 16
_NG = 64
_F = 14

_NC, _NS = 2, 16
_NW = _NC * _NS
_CH = 128
_NE_PAD = 163840
_EPT = _NE_PAD // _NW
_RPT = _EPT // _CH
_NIDXROW = _NE_PAD // _CH
_NN_PAD = 10240
_RO = _NN_PAD // _NS

_EBLK = 2048


@functools.lru_cache(maxsize=1)
def _sc_kernels():
    mesh = plsc.VectorSubcoreMesh(
        core_axis_name="c", subcore_axis_name="s",
        num_cores=_NC, num_subcores=_NS)

    @functools.partial(
        pl.kernel,
        out_type=jax.ShapeDtypeStruct((_NE_PAD, _D), jnp.float32),
        mesh=mesh,
        scratch_types=[
            pltpu.VMEM((_RPT, _CH), jnp.int32),
            pltpu.VMEM((_EPT, _D), jnp.float32),
            pltpu.SemaphoreType.DMA,
        ],
        compiler_params=pltpu.CompilerParams(use_tc_tiling_on_sc=False),
    )
    def sc_gather(x_hbm, src_hbm, out_hbm, idx_v, rows_v, sem):
        wid = lax.axis_index("s") * _NC + lax.axis_index("c")
        pltpu.sync_copy(src_hbm.at[pl.ds(wid * _RPT, _RPT)], idx_v)

        def fire(j, carry):
            pltpu.make_async_copy(
                x_hbm.at[idx_v.at[j]],
                rows_v.at[pl.ds(j * _CH, _CH)], sem).start()
            return carry

        lax.fori_loop(0, _RPT, fire, 0)

        def drain(j, carry):
            pltpu.make_async_copy(
                x_hbm.at[idx_v.at[j]],
                rows_v.at[pl.ds(j * _CH, _CH)], sem).wait()
            return carry

        lax.fori_loop(0, _RPT, drain, 0)
        pltpu.sync_copy(rows_v, out_hbm.at[pl.ds(wid * _EPT, _EPT)])

    @functools.partial(
        pl.kernel,
        out_type=jax.ShapeDtypeStruct((_NC, _NN_PAD, _D), jnp.float32),
        mesh=mesh,
        scratch_types=[
            pltpu.VMEM((_RPT, _CH), jnp.int32),
            pltpu.VMEM((_EPT, _D), jnp.float32),
            pltpu.VMEM_SHARED((_NN_PAD, _D), jnp.float32),
        ],
        compiler_params=pltpu.CompilerParams(use_tc_tiling_on_sc=False),
    )
    def sc_scatter(msg_hbm, dst_hbm, zero_hbm, out_hbm, idx_v, rows_v, acc):
        cid = lax.axis_index("c")
        sid = lax.axis_index("s")
        wid = sid * _NC + cid
        pltpu.sync_copy(dst_hbm.at[pl.ds(wid * _RPT, _RPT)], idx_v)
        pltpu.sync_copy(msg_hbm.at[pl.ds(wid * _EPT, _EPT)], rows_v)

        @pl.when(sid == 0)
        def _():
            pltpu.sync_copy(zero_hbm, acc)

        plsc.subcore_barrier()

        def body(j, carry):
            pltpu.sync_copy(rows_v.at[pl.ds(j * _CH, _CH)],
                            acc.at[idx_v.at[j]], add=True)
            return carry

        lax.fori_loop(0, _RPT, body, 0)
        plsc.subcore_barrier()
        pltpu.sync_copy(acc.at[pl.ds(sid * _RO, _RO)],
                        out_hbm.at[cid, pl.ds(sid * _RO, _RO)])

    @functools.partial(
        pl.kernel,
        out_type=jax.ShapeDtypeStruct((_NC, _NN_PAD, _D), jnp.float32),
        mesh=mesh,
        scratch_types=[
            pltpu.VMEM((_RPT, _CH), jnp.int32),
            pltpu.VMEM((_CH, _D), jnp.float32),
            pltpu.VMEM_SHARED((_NN_PAD, _D), jnp.float32),
        ],
        compiler_params=pltpu.CompilerParams(use_tc_tiling_on_sc=False),
    )
    def sc_count(dst_hbm, ones_hbm, zero_hbm, out_hbm, idx_v, ones_v, acc):
        cid = lax.axis_index("c")
        sid = lax.axis_index("s")
        wid = sid * _NC + cid
        pltpu.sync_copy(dst_hbm.at[pl.ds(wid * _RPT, _RPT)], idx_v)
        pltpu.sync_copy(ones_hbm, ones_v)

        @pl.when(sid == 0)
        def _():
            pltpu.sync_copy(zero_hbm, acc)

        plsc.subcore_barrier()

        def body(j, carry):
            pltpu.sync_copy(ones_v, acc.at[idx_v.at[j]], add=True)
            return carry

        lax.fori_loop(0, _RPT, body, 0)
        plsc.subcore_barrier()
        pltpu.sync_copy(acc.at[pl.ds(sid * _RO, _RO)],
                        out_hbm.at[cid, pl.ds(sid * _RO, _RO)])

    return sc_gather, sc_scatter, sc_count


_HI = lax.Precision.HIGHEST
_BF = jnp.bfloat16


def _mmb(a, b):
    return jnp.dot(a.astype(_BF), b.astype(_BF),
                   preferred_element_type=jnp.float32)


def _lin0_body(x_ref, w_ref, b_ref, out_ref):
    out_ref[...] = jnp.maximum(_mmb(x_ref[...], w_ref[...]) + b_ref[...], 0.0)


def _wf_body(ea_ref, w1_ref, b1_ref, w2_ref, b2_ref, out_ref):
    h = jnp.maximum(_mmb(ea_ref[...], w1_ref[...]) + b1_ref[...], 0.0)
    out_ref[...] = (_mmb(h, w2_ref[...]) + b2_ref[...]).astype(_BF)


def _msg_body(wf_ref, xj_ref, r_ref, s_ref, out_ref):
    p = jnp.dot(xj_ref[...].astype(_BF), r_ref[...].astype(_BF),
                preferred_element_type=jnp.float32)
    out_ref[...] = jnp.dot(p * wf_ref[...].astype(jnp.float32),
                           s_ref[...], precision=_HI,
                           preferred_element_type=jnp.float32)


def _node_body(s0_ref, s1_ref, c0_ref, c1_ref, x_ref, root_ref, cb_ref,
               wir_ref, whr_ref, wiz_ref, whz_ref, win_ref, whn_ref,
               br_ref, bz_ref, bn_ref, bhr_ref, bhz_ref, bhn_ref, out_ref):
    s = s0_ref[...] + s1_ref[...]
    cnt = jnp.maximum(c0_ref[...] + c1_ref[...], 1.0)
    x = x_ref[...]
    m = s / cnt + _mmb(x, root_ref[...]) + cb_ref[...]
    m = jnp.maximum(m, 0.0)
    gr = jax.nn.sigmoid(
        _mmb(m, wir_ref[...]) + br_ref[...]
        + _mmb(x, whr_ref[...]) + bhr_ref[...])
    gz = jax.nn.sigmoid(
        _mmb(m, wiz_ref[...]) + bz_ref[...]
        + _mmb(x, whz_ref[...]) + bhz_ref[...])
    gn = jnp.tanh(
        _mmb(m, win_ref[...]) + bn_ref[...]
        + gr * (_mmb(x, whn_ref[...]) + bhn_ref[...]))
    out_ref[...] = (1.0 - gz) * gn + gz * x


def _s2s_body(x_ref, batch_ref, wq_ref, wr_ref, wh_ref, bg_ref,
              l1q_ref, l1r_ref, l1b_ref, l2_ref, l2b_ref,
              m1_ref, m2_ref, dd_ref, out_ref):
    x = x_ref[...]
    seg = batch_ref[...] == lax.broadcasted_iota(jnp.int32, (_NN, _NG), 1)
    segf = seg.astype(jnp.float32)
    wq = wq_ref[...]
    wr = wr_ref[...]
    wh = wh_ref[...]
    bg = bg_ref[...]
    q = jnp.zeros((_NG, _D), jnp.float32)
    r = jnp.zeros((_NG, _D), jnp.float32)
    h = jnp.zeros((_NG, _D), jnp.float32)
    c = jnp.zeros((_NG, _D), jnp.float32)
    for _ in range(3):
        gates = (_mmb(q, wq) + _mmb(r, wr) + _mmb(h, wh) + bg)
        gi = jax.nn.sigmoid(gates[:, 0 * _D:1 * _D])
        gf = jax.nn.sigmoid(gates[:, 1 * _D:2 * _D])
        gg = jnp.tanh(gates[:, 2 * _D:3 * _D])
        go = jax.nn.sigmoid(gates[:, 3 * _D:4 * _D])
        c = gf * c + gi * gg
        h = go * jnp.tanh(c)
        q = h
        qb = jnp.dot(segf, q, precision=_HI,
                     preferred_element_type=jnp.float32)
        e = jnp.sum(x * qb, axis=1, keepdims=True)
        mx = jnp.max(jnp.where(seg, e, -1e30), axis=0, keepdims=True)
        mb = jnp.sum(segf * mx, axis=1, keepdims=True)
        a = jnp.exp(e - mb)
        denom = lax.dot_general(segf, a, (((0,), (0,)), ((), ())),
                                precision=_HI,
                                preferred_element_type=jnp.float32)
        db = jnp.dot(segf, denom, precision=_HI,
                     preferred_element_type=jnp.float32)
        a = a / (db + 1e-16)
        r = lax.dot_general(segf, a * x, (((0,), (0,)), ((), ())),
                            precision=_HI,
                            preferred_element_type=jnp.float32)
    o = jnp.maximum(_mmb(q, l1q_ref[...]) + _mmb(r, l1r_ref[...])
                    + l1b_ref[...], 0.0)
    dd = dd_ref[...] > 0.5
    o = jnp.where(dd, o * (m1_ref[...] * 2.0), o)
    o = _mmb(o, l2_ref[...]) + l2b_ref[...]
    o = jnp.where(dd, o * (m2_ref[...] * 2.0), o)
    out_ref[...] = o


def _tc_call(body, out_shape, *args):
    return pl.pallas_call(body, out_shape=out_shape)(*args)


def kernel(x, edge_index, edge_attr, batch, do_dropout, lin0_W, lin0_b,
           h1_W, h1_b, h2_W, h2_b, conv_root, conv_bias,
           gru_W_ih, gru_W_hh, gru_b_ih, gru_b_hh,
           lstm_W_ih, lstm_W_hh, lstm_b_ih, lstm_b_hh,
           lin1_W, lin1_b, lin2_W, lin2_b):
    f32 = jnp.float32
    x = x.astype(f32)
    ei = edge_index.astype(jnp.int32)
    npad = _NE_PAD - _NE
    src2d = jnp.concatenate(
        [ei[0], jnp.zeros((npad,), jnp.int32)]).reshape(_NIDXROW, _CH)
    dst2d = jnp.concatenate(
        [ei[1], jnp.full((npad,), _NN, jnp.int32)]).reshape(_NIDXROW, _CH)
    ea_p = jnp.pad(edge_attr.astype(f32), ((0, npad), (0, 0)))
    zeros_acc = jnp.zeros((_NN_PAD, _D), f32)
    ones_ch = jnp.ones((_CH, _D), f32)

    rmat = jnp.kron(jnp.eye(_D, dtype=f32), jnp.ones((1, _D), f32))
    smat = jnp.tile(jnp.eye(_D, dtype=f32), (_D, 1))

    w1t = h1_W.astype(f32).T
    b1 = h1_b.astype(f32).reshape(1, -1)
    w2t = h2_W.astype(f32).T
    b2 = h2_b.astype(f32).reshape(1, -1)
    l0t = lin0_W.astype(f32).T
    l0b = lin0_b.astype(f32).reshape(1, -1)
    root = conv_root.astype(f32)
    cb = conv_bias.astype(f32).reshape(1, -1)

    giT = gru_W_ih.astype(f32).T
    ghT = gru_W_hh.astype(f32).T
    wir, wiz, win = giT[:, :_D], giT[:, _D:2 * _D], giT[:, 2 * _D:]
    whr, whz, whn = ghT[:, :_D], ghT[:, _D:2 * _D], ghT[:, 2 * _D:]
    bir, biz, bin_ = (gru_b_ih[:_D].reshape(1, -1).astype(f32),
                      gru_b_ih[_D:2 * _D].reshape(1, -1).astype(f32),
                      gru_b_ih[2 * _D:].reshape(1, -1).astype(f32))
    bhr, bhz, bhn = (gru_b_hh[:_D].reshape(1, -1).astype(f32),
                     gru_b_hh[_D:2 * _D].reshape(1, -1).astype(f32),
                     gru_b_hh[2 * _D:].reshape(1, -1).astype(f32))

    liT = lstm_W_ih.astype(f32).T
    wq = liT[:_D, :]
    wr_ = liT[_D:, :]
    wh = lstm_W_hh.astype(f32).T
    bg = (lstm_b_ih + lstm_b_hh).astype(f32).reshape(1, -1)

    l1t = lin1_W.astype(f32).T
    l1q, l1r = l1t[:_D, :], l1t[_D:, :]
    l1b = lin1_b.astype(f32).reshape(1, -1)
    l2t = lin2_W.astype(f32).T
    l2b = lin2_b.astype(f32).reshape(1, -1)
    batch2d = batch.astype(jnp.int32).reshape(_NN, 1)
    mask1 = jax.random.bernoulli(
        jax.random.key(123), 0.5, (_NG, _D)).astype(f32)
    mask2 = jax.random.bernoulli(
        jax.random.key(456), 0.5, (_NG, 1)).astype(f32)
    dd = jnp.asarray(do_dropout, f32).reshape(1, 1)

    sc_gather, sc_scatter, sc_count = _sc_kernels()

    nblk = 1000
    ngrid = _NN // nblk
    full = lambda shape: pl.BlockSpec(shape, lambda i: (0, 0))
    rows = lambda lanes: pl.BlockSpec((nblk, lanes), lambda i: (i, 0))

    h = pl.pallas_call(
        _lin0_body,
        grid=(ngrid,),
        in_specs=[rows(_F), full((_F, _D)), full((1, _D))],
        out_specs=rows(_D),
        out_shape=jax.ShapeDtypeStruct((_NN, _D), f32),
    )(x, l0t, l0b)

    cnt_part = sc_count(dst2d, ones_ch, zeros_acc)
    c0 = cnt_part[0, :_NN]
    c1 = cnt_part[1, :_NN]

    grid = _NE_PAD // _EBLK
    wf_call = pl.pallas_call(
        _wf_body,
        grid=(grid,),
        in_specs=[
            pl.BlockSpec((_EBLK, 4), lambda i: (i, 0)),
            pl.BlockSpec((4, 128), lambda i: (0, 0)),
            pl.BlockSpec((1, 128), lambda i: (0, 0)),
            pl.BlockSpec((128, 256), lambda i: (0, 0)),
            pl.BlockSpec((1, 256), lambda i: (0, 0)),
        ],
        out_specs=pl.BlockSpec((_EBLK, 256), lambda i: (i, 0)),
        out_shape=jax.ShapeDtypeStruct((_NE_PAD, 256), jnp.bfloat16),
    )
    wf = wf_call(ea_p, w1t, b1, w2t, b2)
    msg_call = pl.pallas_call(
        _msg_body,
        grid=(grid,),
        in_specs=[
            pl.BlockSpec((_EBLK, 256), lambda i: (i, 0)),
            pl.BlockSpec((_EBLK, _D), lambda i: (i, 0)),
            pl.BlockSpec((_D, 256), lambda i: (0, 0)),
            pl.BlockSpec((256, _D), lambda i: (0, 0)),
        ],
        out_specs=pl.BlockSpec((_EBLK, _D), lambda i: (i, 0)),
        out_shape=jax.ShapeDtypeStruct((_NE_PAD, _D), f32),
    )

    node_call = pl.pallas_call(
        _node_body,
        grid=(ngrid,),
        in_specs=[rows(_D)] * 5 + [full((_D, _D)), full((1, _D))]
        + [full((_D, _D))] * 6 + [full((1, _D))] * 6,
        out_specs=rows(_D),
        out_shape=jax.ShapeDtypeStruct((_NN, _D), f32),
    )

    out = h
    for _ in range(3):
        xj = sc_gather(out, src2d)
        msg = msg_call(wf, xj, rmat, smat)
        s_part = sc_scatter(msg, dst2d, zeros_acc)
        out = node_call(
            s_part[0, :_NN], s_part[1, :_NN], c0, c1, out, root, cb,
            wir, whr, wiz, whz, win, whn, bir, biz, bin_, bhr, bhz, bhn)

    o = _tc_call(
        _s2s_body, jax.ShapeDtypeStruct((_NG, 1), f32),
        out, batch2d, wq, wr_, wh, bg, l1q, l1r, l1b, l2t, l2b,
        mask1, mask2, dd)
    return o.reshape(-1)


# speedup vs baseline: 1.1718x; 1.1718x over previous
import functools

import jax
import jax.numpy as jnp
from jax import lax
from jax.experimental import pallas as pl
from jax.experimental.pallas import tpu as pltpu
from jax.experimental.pallas import tpu_sc as plsc

_NN = 10000
_NE = 160000
_D = 16
_NG = 64
_F = 14

_NC, _NS = 2, 16
_NW = _NC * _NS
_CH = 128
_NE_PAD = 163840
_EPT = _NE_PAD // _NW
_RPT = _EPT // _CH
_NIDXROW = _NE_PAD // _CH
_NN_PAD = 10240
_RO = _NN_PAD // _NS

_EBLK = 2048


@functools.lru_cache(maxsize=1)
def _sc_kernels():
    mesh = plsc.VectorSubcoreMesh(
        core_axis_name="c", subcore_axis_name="s",
        num_cores=_NC, num_subcores=_NS)

    @functools.partial(
        pl.kernel,
        out_type=jax.ShapeDtypeStruct((_NE_PAD, _D), jnp.float32),
        mesh=mesh,
        scratch_types=[
            pltpu.VMEM((_RPT, _CH), jnp.int32),
            pltpu.VMEM((_EPT, _D), jnp.float32),
            pltpu.SemaphoreType.DMA,
        ],
        compiler_params=pltpu.CompilerParams(use_tc_tiling_on_sc=False),
    )
    def sc_gather(x_hbm, src_hbm, out_hbm, idx_v, rows_v, sem):
        wid = lax.axis_index("s") * _NC + lax.axis_index("c")
        pltpu.sync_copy(src_hbm.at[pl.ds(wid * _RPT, _RPT)], idx_v)

        def fire(j, carry):
            pltpu.make_async_copy(
                x_hbm.at[idx_v.at[j]],
                rows_v.at[pl.ds(j * _CH, _CH)], sem).start()
            return carry

        lax.fori_loop(0, _RPT, fire, 0)

        def drain(j, carry):
            pltpu.make_async_copy(
                x_hbm.at[idx_v.at[j]],
                rows_v.at[pl.ds(j * _CH, _CH)], sem).wait()
            return carry

        lax.fori_loop(0, _RPT, drain, 0)
        pltpu.sync_copy(rows_v, out_hbm.at[pl.ds(wid * _EPT, _EPT)])

    @functools.partial(
        pl.kernel,
        out_type=jax.ShapeDtypeStruct((_NC, _NN_PAD, _D), jnp.float32),
        mesh=mesh,
        scratch_types=[
            pltpu.VMEM((_RPT, _CH), jnp.int32),
            pltpu.VMEM((_EPT, _D), jnp.float32),
            pltpu.VMEM_SHARED((_NN_PAD, _D), jnp.float32),
        ],
        compiler_params=pltpu.CompilerParams(use_tc_tiling_on_sc=False),
    )
    def sc_scatter(msg_hbm, dst_hbm, zero_hbm, out_hbm, idx_v, rows_v, acc):
        cid = lax.axis_index("c")
        sid = lax.axis_index("s")
        wid = sid * _NC + cid
        pltpu.sync_copy(dst_hbm.at[pl.ds(wid * _RPT, _RPT)], idx_v)
        pltpu.sync_copy(msg_hbm.at[pl.ds(wid * _EPT, _EPT)], rows_v)

        @pl.when(sid == 0)
        def _():
            pltpu.sync_copy(zero_hbm, acc)

        plsc.subcore_barrier()

        def body(j, carry):
            pltpu.sync_copy(rows_v.at[pl.ds(j * _CH, _CH)],
                            acc.at[idx_v.at[j]], add=True)
            return carry

        lax.fori_loop(0, _RPT, body, 0)
        plsc.subcore_barrier()
        pltpu.sync_copy(acc.at[pl.ds(sid * _RO, _RO)],
                        out_hbm.at[cid, pl.ds(sid * _RO, _RO)])

    @functools.partial(
        pl.kernel,
        out_type=jax.ShapeDtypeStruct((_NC, _NN_PAD, _D), jnp.float32),
        mesh=mesh,
        scratch_types=[
            pltpu.VMEM((_RPT, _CH), jnp.int32),
            pltpu.VMEM((_CH, _D), jnp.float32),
            pltpu.VMEM_SHARED((_NN_PAD, _D), jnp.float32),
        ],
        compiler_params=pltpu.CompilerParams(use_tc_tiling_on_sc=False),
    )
    def sc_count(dst_hbm, ones_hbm, zero_hbm, out_hbm, idx_v, ones_v, acc):
        cid = lax.axis_index("c")
        sid = lax.axis_index("s")
        wid = sid * _NC + cid
        pltpu.sync_copy(dst_hbm.at[pl.ds(wid * _RPT, _RPT)], idx_v)
        pltpu.sync_copy(ones_hbm, ones_v)

        @pl.when(sid == 0)
        def _():
            pltpu.sync_copy(zero_hbm, acc)

        plsc.subcore_barrier()

        def body(j, carry):
            pltpu.sync_copy(ones_v, acc.at[idx_v.at[j]], add=True)
            return carry

        lax.fori_loop(0, _RPT, body, 0)
        plsc.subcore_barrier()
        pltpu.sync_copy(acc.at[pl.ds(sid * _RO, _RO)],
                        out_hbm.at[cid, pl.ds(sid * _RO, _RO)])

    return sc_gather, sc_scatter, sc_count


_HI = lax.Precision.HIGHEST
_BF = jnp.bfloat16


def _mmb(a, b):
    return jnp.dot(a.astype(_BF), b.astype(_BF),
                   preferred_element_type=jnp.float32)


def _lin0_body(x_ref, w_ref, b_ref, out_ref):
    out_ref[...] = jnp.maximum(_mmb(x_ref[...], w_ref[...]) + b_ref[...], 0.0)


def _wf_body(ea_ref, w1_ref, b1_ref, w2_ref, b2_ref, out_ref):
    h = jnp.maximum(_mmb(ea_ref[...], w1_ref[...]) + b1_ref[...], 0.0)
    out_ref[...] = (_mmb(h, w2_ref[...]) + b2_ref[...]).astype(_BF)


def _msg_body(wf_ref, xj_ref, r_ref, out_ref):
    p = jnp.dot(xj_ref[...].astype(_BF), r_ref[...].astype(_BF),
                preferred_element_type=jnp.float32)
    a = p * wf_ref[...].astype(jnp.float32)
    a = a[:, :128] + a[:, 128:]
    a = a[:, :64] + a[:, 64:]
    a = a[:, :32] + a[:, 32:]
    out_ref[...] = a[:, :16] + a[:, 16:]


def _node_body(s0_ref, s1_ref, c0_ref, c1_ref, x_ref, root_ref, cb_ref,
               wir_ref, whr_ref, wiz_ref, whz_ref, win_ref, whn_ref,
               br_ref, bz_ref, bn_ref, bhr_ref, bhz_ref, bhn_ref, out_ref):
    s = s0_ref[...] + s1_ref[...]
    cnt = jnp.maximum(c0_ref[...] + c1_ref[...], 1.0)
    x = x_ref[...]
    m = s / cnt + _mmb(x, root_ref[...]) + cb_ref[...]
    m = jnp.maximum(m, 0.0)
    gr = jax.nn.sigmoid(
        _mmb(m, wir_ref[...]) + br_ref[...]
        + _mmb(x, whr_ref[...]) + bhr_ref[...])
    gz = jax.nn.sigmoid(
        _mmb(m, wiz_ref[...]) + bz_ref[...]
        + _mmb(x, whz_ref[...]) + bhz_ref[...])
    gn = jnp.tanh(
        _mmb(m, win_ref[...]) + bn_ref[...]
        + gr * (_mmb(x, whn_ref[...]) + bhn_ref[...]))
    out_ref[...] = (1.0 - gz) * gn + gz * x


def _s2s_body(x_ref, batch_ref, wq_ref, wr_ref, wh_ref, bg_ref,
              l1q_ref, l1r_ref, l1b_ref, l2_ref, l2b_ref,
              m1_ref, m2_ref, dd_ref, out_ref):
    x = x_ref[...]
    seg = batch_ref[...] == lax.broadcasted_iota(jnp.int32, (_NN, _NG), 1)
    segf = seg.astype(jnp.float32)
    wq = wq_ref[...]
    wr = wr_ref[...]
    wh = wh_ref[...]
    bg = bg_ref[...]
    q = jnp.zeros((_NG, _D), jnp.float32)
    r = jnp.zeros((_NG, _D), jnp.float32)
    h = jnp.zeros((_NG, _D), jnp.float32)
    c = jnp.zeros((_NG, _D), jnp.float32)
    for _ in range(3):
        gates = (_mmb(q, wq) + _mmb(r, wr) + _mmb(h, wh) + bg)
        gi = jax.nn.sigmoid(gates[:, 0 * _D:1 * _D])
        gf = jax.nn.sigmoid(gates[:, 1 * _D:2 * _D])
        gg = jnp.tanh(gates[:, 2 * _D:3 * _D])
        go = jax.nn.sigmoid(gates[:, 3 * _D:4 * _D])
        c = gf * c + gi * gg
        h = go * jnp.tanh(c)
        q = h
        qb = jnp.dot(segf, q, precision=_HI,
                     preferred_element_type=jnp.float32)
        e = jnp.sum(x * qb, axis=1, keepdims=True)
        mx = jnp.max(jnp.where(seg, e, -1e30), axis=0, keepdims=True)
        mb = jnp.sum(segf * mx, axis=1, keepdims=True)
        a = jnp.exp(e - mb)
        denom = lax.dot_general(segf, a, (((0,), (0,)), ((), ())),
                                precision=_HI,
                                preferred_element_type=jnp.float32)
        db = jnp.dot(segf, denom, precision=_HI,
                     preferred_element_type=jnp.float32)
        a = a / (db + 1e-16)
        r = lax.dot_general(segf, a * x, (((0,), (0,)), ((), ())),
                            precision=_HI,
                            preferred_element_type=jnp.float32)
    o = jnp.maximum(_mmb(q, l1q_ref[...]) + _mmb(r, l1r_ref[...])
                    + l1b_ref[...], 0.0)
    dd = dd_ref[...] > 0.5
    o = jnp.where(dd, o * (m1_ref[...] * 2.0), o)
    o = _mmb(o, l2_ref[...]) + l2b_ref[...]
    o = jnp.where(dd, o * (m2_ref[...] * 2.0), o)
    out_ref[...] = o


def _tc_call(body, out_shape, *args):
    return pl.pallas_call(body, out_shape=out_shape)(*args)


def kernel(x, edge_index, edge_attr, batch, do_dropout, lin0_W, lin0_b,
           h1_W, h1_b, h2_W, h2_b, conv_root, conv_bias,
           gru_W_ih, gru_W_hh, gru_b_ih, gru_b_hh,
           lstm_W_ih, lstm_W_hh, lstm_b_ih, lstm_b_hh,
           lin1_W, lin1_b, lin2_W, lin2_b):
    f32 = jnp.float32
    x = x.astype(f32)
    ei = edge_index.astype(jnp.int32)
    npad = _NE_PAD - _NE
    src2d = jnp.concatenate(
        [ei[0], jnp.zeros((npad,), jnp.int32)]).reshape(_NIDXROW, _CH)
    dst2d = jnp.concatenate(
        [ei[1], jnp.full((npad,), _NN, jnp.int32)]).reshape(_NIDXROW, _CH)
    ea_p = jnp.pad(edge_attr.astype(f32), ((0, npad), (0, 0)))
    zeros_acc = jnp.zeros((_NN_PAD, _D), f32)
    ones_ch = jnp.ones((_CH, _D), f32)

    rmat = jnp.kron(jnp.eye(_D, dtype=f32), jnp.ones((1, _D), f32))
    smat = jnp.tile(jnp.eye(_D, dtype=f32), (_D, 1))

    w1t = h1_W.astype(f32).T
    b1 = h1_b.astype(f32).reshape(1, -1)
    w2t = h2_W.astype(f32).T
    b2 = h2_b.astype(f32).reshape(1, -1)
    l0t = lin0_W.astype(f32).T
    l0b = lin0_b.astype(f32).reshape(1, -1)
    root = conv_root.astype(f32)
    cb = conv_bias.astype(f32).reshape(1, -1)

    giT = gru_W_ih.astype(f32).T
    ghT = gru_W_hh.astype(f32).T
    wir, wiz, win = giT[:, :_D], giT[:, _D:2 * _D], giT[:, 2 * _D:]
    whr, whz, whn = ghT[:, :_D], ghT[:, _D:2 * _D], ghT[:, 2 * _D:]
    bir, biz, bin_ = (gru_b_ih[:_D].reshape(1, -1).astype(f32),
                      gru_b_ih[_D:2 * _D].reshape(1, -1).astype(f32),
                      gru_b_ih[2 * _D:].reshape(1, -1).astype(f32))
    bhr, bhz, bhn = (gru_b_hh[:_D].reshape(1, -1).astype(f32),
                     gru_b_hh[_D:2 * _D].reshape(1, -1).astype(f32),
                     gru_b_hh[2 * _D:].reshape(1, -1).astype(f32))

    liT = lstm_W_ih.astype(f32).T
    wq = liT[:_D, :]
    wr_ = liT[_D:, :]
    wh = lstm_W_hh.astype(f32).T
    bg = (lstm_b_ih + lstm_b_hh).astype(f32).reshape(1, -1)

    l1t = lin1_W.astype(f32).T
    l1q, l1r = l1t[:_D, :], l1t[_D:, :]
    l1b = lin1_b.astype(f32).reshape(1, -1)
    l2t = lin2_W.astype(f32).T
    l2b = lin2_b.astype(f32).reshape(1, -1)
    batch2d = batch.astype(jnp.int32).reshape(_NN, 1)
    mask1 = jax.random.bernoulli(
        jax.random.key(123), 0.5, (_NG, _D)).astype(f32)
    mask2 = jax.random.bernoulli(
        jax.random.key(456), 0.5, (_NG, 1)).astype(f32)
    dd = jnp.asarray(do_dropout, f32).reshape(1, 1)

    sc_gather, sc_scatter, sc_count = _sc_kernels()

    nblk = 1000
    ngrid = _NN // nblk
    full = lambda shape: pl.BlockSpec(shape, lambda i: (0, 0))
    rows = lambda lanes: pl.BlockSpec((nblk, lanes), lambda i: (i, 0))

    h = pl.pallas_call(
        _lin0_body,
        grid=(ngrid,),
        in_specs=[rows(_F), full((_F, _D)), full((1, _D))],
        out_specs=rows(_D),
        out_shape=jax.ShapeDtypeStruct((_NN, _D), f32),
    )(x, l0t, l0b)

    cnt_part = sc_count(dst2d, ones_ch, zeros_acc)
    c0 = cnt_part[0, :_NN]
    c1 = cnt_part[1, :_NN]

    grid = _NE_PAD // _EBLK
    wf_call = pl.pallas_call(
        _wf_body,
        grid=(grid,),
        in_specs=[
            pl.BlockSpec((_EBLK, 4), lambda i: (i, 0)),
            pl.BlockSpec((4, 128), lambda i: (0, 0)),
            pl.BlockSpec((1, 128), lambda i: (0, 0)),
            pl.BlockSpec((128, 256), lambda i: (0, 0)),
            pl.BlockSpec((1, 256), lambda i: (0, 0)),
        ],
        out_specs=pl.BlockSpec((_EBLK, 256), lambda i: (i, 0)),
        out_shape=jax.ShapeDtypeStruct((_NE_PAD, 256), jnp.bfloat16),
    )
    wf = wf_call(ea_p, w1t, b1, w2t, b2)
    msg_call = pl.pallas_call(
        _msg_body,
        grid=(grid,),
        in_specs=[
            pl.BlockSpec((_EBLK, 256), lambda i: (i, 0)),
            pl.BlockSpec((_EBLK, _D), lambda i: (i, 0)),
            pl.BlockSpec((_D, 256), lambda i: (0, 0)),
        ],
        out_specs=pl.BlockSpec((_EBLK, _D), lambda i: (i, 0)),
        out_shape=jax.ShapeDtypeStruct((_NE_PAD, _D), f32),
    )

    node_call = pl.pallas_call(
        _node_body,
        grid=(ngrid,),
        in_specs=[rows(_D)] * 5 + [full((_D, _D)), full((1, _D))]
        + [full((_D, _D))] * 6 + [full((1, _D))] * 6,
        out_specs=rows(_D),
        out_shape=jax.ShapeDtypeStruct((_NN, _D), f32),
    )

    out = h
    for _ in range(3):
        xj = sc_gather(out, src2d)
        msg = msg_call(wf, xj, rmat)
        s_part = sc_scatter(msg, dst2d, zeros_acc)
        out = node_call(
            s_part[0, :_NN], s_part[1, :_NN], c0, c1, out, root, cb,
            wir, whr, wiz, whz, win, whn, bir, biz, bin_, bhr, bhz, bhn)

    o = _tc_call(
        _s2s_body, jax.ShapeDtypeStruct((_NG, 1), f32),
        out, batch2d, wq, wr_, wh, bg, l1q, l1r, l1b, l2t, l2b,
        mask1, mask2, dd)
    return o.reshape(-1)


# speedup vs baseline: 1.2207x; 1.0417x over previous
import functools

import jax
import jax.numpy as jnp
from jax import lax
from jax.experimental import pallas as pl
from jax.experimental.pallas import tpu as pltpu
from jax.experimental.pallas import tpu_sc as plsc

_NN = 10000
_NE = 160000
_D = 16
_NG = 64
_F = 14

_NC, _NS = 2, 16
_NW = _NC * _NS
_CH = 128
_NE_PAD = 163840
_EPT = _NE_PAD // _NW
_RPT = _EPT // _CH
_NIDXROW = _NE_PAD // _CH
_NN_PAD = 10240
_RO = _NN_PAD // _NS

_EBLK = 2048


@functools.lru_cache(maxsize=1)
def _sc_kernels():
    mesh = plsc.VectorSubcoreMesh(
        core_axis_name="c", subcore_axis_name="s",
        num_cores=_NC, num_subcores=_NS)

    @functools.partial(
        pl.kernel,
        out_type=jax.ShapeDtypeStruct((_NE_PAD, _D), jnp.float32),
        mesh=mesh,
        scratch_types=[
            pltpu.VMEM((_RPT, _CH), jnp.int32),
            pltpu.VMEM((_EPT, _D), jnp.float32),
            pltpu.SemaphoreType.DMA,
        ],
        compiler_params=pltpu.CompilerParams(use_tc_tiling_on_sc=False),
    )
    def sc_gather(x_hbm, src_hbm, out_hbm, idx_v, rows_v, sem):
        wid = lax.axis_index("s") * _NC + lax.axis_index("c")
        pltpu.sync_copy(src_hbm.at[pl.ds(wid * _RPT, _RPT)], idx_v)

        def fire(j, carry):
            pltpu.make_async_copy(
                x_hbm.at[idx_v.at[j]],
                rows_v.at[pl.ds(j * _CH, _CH)], sem).start()
            return carry

        lax.fori_loop(0, _RPT, fire, 0)

        def drain(j, carry):
            pltpu.make_async_copy(
                x_hbm.at[idx_v.at[j]],
                rows_v.at[pl.ds(j * _CH, _CH)], sem).wait()
            return carry

        lax.fori_loop(0, _RPT, drain, 0)
        pltpu.sync_copy(rows_v, out_hbm.at[pl.ds(wid * _EPT, _EPT)])

    @functools.partial(
        pl.kernel,
        out_type=jax.ShapeDtypeStruct((_NC, _NN_PAD, _D), jnp.float32),
        mesh=mesh,
        scratch_types=[
            pltpu.VMEM((_RPT, _CH), jnp.int32),
            pltpu.VMEM((_EPT, _D), jnp.float32),
            pltpu.VMEM_SHARED((_NN_PAD, _D), jnp.float32),
        ],
        compiler_params=pltpu.CompilerParams(use_tc_tiling_on_sc=False),
    )
    def sc_scatter(msg_hbm, dst_hbm, zero_hbm, out_hbm, idx_v, rows_v, acc):
        cid = lax.axis_index("c")
        sid = lax.axis_index("s")
        wid = sid * _NC + cid
        pltpu.sync_copy(dst_hbm.at[pl.ds(wid * _RPT, _RPT)], idx_v)
        pltpu.sync_copy(msg_hbm.at[pl.ds(wid * _EPT, _EPT)], rows_v)

        @pl.when(sid == 0)
        def _():
            pltpu.sync_copy(zero_hbm, acc)

        plsc.subcore_barrier()

        def body(j, carry):
            pltpu.sync_copy(rows_v.at[pl.ds(j * _CH, _CH)],
                            acc.at[idx_v.at[j]], add=True)
            return carry

        lax.fori_loop(0, _RPT, body, 0)
        plsc.subcore_barrier()
        pltpu.sync_copy(acc.at[pl.ds(sid * _RO, _RO)],
                        out_hbm.at[cid, pl.ds(sid * _RO, _RO)])

    @functools.partial(
        pl.kernel,
        out_type=jax.ShapeDtypeStruct((_NC, _NN_PAD, _D), jnp.float32),
        mesh=mesh,
        scratch_types=[
            pltpu.VMEM((_RPT, _CH), jnp.int32),
            pltpu.VMEM((_CH, _D), jnp.float32),
            pltpu.VMEM_SHARED((_NN_PAD, _D), jnp.float32),
        ],
        compiler_params=pltpu.CompilerParams(use_tc_tiling_on_sc=False),
    )
    def sc_count(dst_hbm, ones_hbm, zero_hbm, out_hbm, idx_v, ones_v, acc):
        cid = lax.axis_index("c")
        sid = lax.axis_index("s")
        wid = sid * _NC + cid
        pltpu.sync_copy(dst_hbm.at[pl.ds(wid * _RPT, _RPT)], idx_v)
        pltpu.sync_copy(ones_hbm, ones_v)

        @pl.when(sid == 0)
        def _():
            pltpu.sync_copy(zero_hbm, acc)

        plsc.subcore_barrier()

        def body(j, carry):
            pltpu.sync_copy(ones_v, acc.at[idx_v.at[j]], add=True)
            return carry

        lax.fori_loop(0, _RPT, body, 0)
        plsc.subcore_barrier()
        pltpu.sync_copy(acc.at[pl.ds(sid * _RO, _RO)],
                        out_hbm.at[cid, pl.ds(sid * _RO, _RO)])

    return sc_gather, sc_scatter, sc_count


_HI = lax.Precision.HIGHEST
_BF = jnp.bfloat16


def _mmb(a, b):
    return jnp.dot(a.astype(_BF), b.astype(_BF),
                   preferred_element_type=jnp.float32)


def _lin0_body(x_ref, w_ref, b_ref, out_ref):
    out_ref[...] = jnp.maximum(_mmb(x_ref[...], w_ref[...]) + b_ref[...], 0.0)


def _wf_body(ea_ref, w1_ref, b1_ref, w2_ref, b2_ref, out_ref):
    h = jnp.maximum(_mmb(ea_ref[...], w1_ref[...]) + b1_ref[...], 0.0)
    out_ref[...] = (_mmb(h, w2_ref[...]) + b2_ref[...]).astype(_BF)


def _msg_body(wf_ref, xj_ref, r_ref, out_ref):
    p = jnp.dot(xj_ref[...].astype(_BF), r_ref[...].astype(_BF),
                preferred_element_type=jnp.float32)
    a = p * wf_ref[...].astype(jnp.float32)
    a = a[:, :128] + a[:, 128:]
    a = a[:, :64] + a[:, 64:]
    a = a[:, :32] + a[:, 32:]
    out_ref[...] = a[:, :16] + a[:, 16:]


def _node_body(s0_ref, s1_ref, c0_ref, c1_ref, x_ref, root_ref, cb_ref,
               wir_ref, whr_ref, wiz_ref, whz_ref, win_ref, whn_ref,
               br_ref, bz_ref, bn_ref, bhr_ref, bhz_ref, bhn_ref, out_ref):
    s = s0_ref[...] + s1_ref[...]
    cnt = jnp.maximum(c0_ref[...] + c1_ref[...], 1.0)
    x = x_ref[...]
    m = s / cnt + _mmb(x, root_ref[...]) + cb_ref[...]
    m = jnp.maximum(m, 0.0)
    gr = jax.nn.sigmoid(
        _mmb(m, wir_ref[...]) + br_ref[...]
        + _mmb(x, whr_ref[...]) + bhr_ref[...])
    gz = jax.nn.sigmoid(
        _mmb(m, wiz_ref[...]) + bz_ref[...]
        + _mmb(x, whz_ref[...]) + bhz_ref[...])
    gn = jnp.tanh(
        _mmb(m, win_ref[...]) + bn_ref[...]
        + gr * (_mmb(x, whn_ref[...]) + bhn_ref[...]))
    out_ref[...] = (1.0 - gz) * gn + gz * x


def _s2s_body(x_ref, batch_ref, wq_ref, wr_ref, wh_ref, bg_ref,
              l1q_ref, l1r_ref, l1b_ref, l2_ref, l2b_ref,
              m1_ref, m2_ref, dd_ref, out_ref):
    x = x_ref[...]
    seg = batch_ref[...] == lax.broadcasted_iota(jnp.int32, (_NN, _NG), 1)
    segf = seg.astype(jnp.float32)
    wq = wq_ref[...]
    wr = wr_ref[...]
    wh = wh_ref[...]
    bg = bg_ref[...]
    q = jnp.zeros((_NG, _D), jnp.float32)
    r = jnp.zeros((_NG, _D), jnp.float32)
    h = jnp.zeros((_NG, _D), jnp.float32)
    c = jnp.zeros((_NG, _D), jnp.float32)
    for _ in range(3):
        gates = (_mmb(q, wq) + _mmb(r, wr) + _mmb(h, wh) + bg)
        gi = jax.nn.sigmoid(gates[:, 0 * _D:1 * _D])
        gf = jax.nn.sigmoid(gates[:, 1 * _D:2 * _D])
        gg = jnp.tanh(gates[:, 2 * _D:3 * _D])
        go = jax.nn.sigmoid(gates[:, 3 * _D:4 * _D])
        c = gf * c + gi * gg
        h = go * jnp.tanh(c)
        q = h
        qb = jnp.dot(segf, q, precision=_HI,
                     preferred_element_type=jnp.float32)
        e = jnp.sum(x * qb, axis=1, keepdims=True)
        mx = jnp.max(jnp.where(seg, e, -1e30), axis=0, keepdims=True)
        mb = jnp.sum(segf * mx, axis=1, keepdims=True)
        a = jnp.exp(e - mb)
        denom = lax.dot_general(segf, a, (((0,), (0,)), ((), ())),
                                precision=_HI,
                                preferred_element_type=jnp.float32)
        db = jnp.dot(segf, denom, precision=_HI,
                     preferred_element_type=jnp.float32)
        a = a / (db + 1e-16)
        r = lax.dot_general(segf, a * x, (((0,), (0,)), ((), ())),
                            precision=_HI,
                            preferred_element_type=jnp.float32)
    o = jnp.maximum(_mmb(q, l1q_ref[...]) + _mmb(r, l1r_ref[...])
                    + l1b_ref[...], 0.0)
    dd = dd_ref[...] > 0.5
    o = jnp.where(dd, o * (m1_ref[...] * 2.0), o)
    o = _mmb(o, l2_ref[...]) + l2b_ref[...]
    o = jnp.where(dd, o * (m2_ref[...] * 2.0), o)
    out_ref[...] = o


def _tc_call(body, out_shape, *args):
    return pl.pallas_call(body, out_shape=out_shape)(*args)


def kernel(x, edge_index, edge_attr, batch, do_dropout, lin0_W, lin0_b,
           h1_W, h1_b, h2_W, h2_b, conv_root, conv_bias,
           gru_W_ih, gru_W_hh, gru_b_ih, gru_b_hh,
           lstm_W_ih, lstm_W_hh, lstm_b_ih, lstm_b_hh,
           lin1_W, lin1_b, lin2_W, lin2_b):
    f32 = jnp.float32
    x = x.astype(f32)
    ei = edge_index.astype(jnp.int32)
    npad = _NE_PAD - _NE
    src2d = jnp.concatenate(
        [ei[0], jnp.zeros((npad,), jnp.int32)]).reshape(_NIDXROW, _CH)
    dst2d = jnp.concatenate(
        [ei[1], jnp.full((npad,), _NN, jnp.int32)]).reshape(_NIDXROW, _CH)
    ea_p = jnp.pad(edge_attr.astype(jnp.bfloat16), ((0, npad), (0, 0)))
    zeros_acc = jnp.zeros((_NN_PAD, _D), f32)
    ones_ch = jnp.ones((_CH, _D), f32)

    rmat = jnp.kron(jnp.eye(_D, dtype=f32), jnp.ones((1, _D), f32))
    smat = jnp.tile(jnp.eye(_D, dtype=f32), (_D, 1))

    w1t = h1_W.astype(f32).T
    b1 = h1_b.astype(f32).reshape(1, -1)
    w2t = h2_W.astype(f32).T
    b2 = h2_b.astype(f32).reshape(1, -1)
    l0t = lin0_W.astype(f32).T
    l0b = lin0_b.astype(f32).reshape(1, -1)
    root = conv_root.astype(f32)
    cb = conv_bias.astype(f32).reshape(1, -1)

    giT = gru_W_ih.astype(f32).T
    ghT = gru_W_hh.astype(f32).T
    wir, wiz, win = giT[:, :_D], giT[:, _D:2 * _D], giT[:, 2 * _D:]
    whr, whz, whn = ghT[:, :_D], ghT[:, _D:2 * _D], ghT[:, 2 * _D:]
    bir, biz, bin_ = (gru_b_ih[:_D].reshape(1, -1).astype(f32),
                      gru_b_ih[_D:2 * _D].reshape(1, -1).astype(f32),
                      gru_b_ih[2 * _D:].reshape(1, -1).astype(f32))
    bhr, bhz, bhn = (gru_b_hh[:_D].reshape(1, -1).astype(f32),
                     gru_b_hh[_D:2 * _D].reshape(1, -1).astype(f32),
                     gru_b_hh[2 * _D:].reshape(1, -1).astype(f32))

    liT = lstm_W_ih.astype(f32).T
    wq = liT[:_D, :]
    wr_ = liT[_D:, :]
    wh = lstm_W_hh.astype(f32).T
    bg = (lstm_b_ih + lstm_b_hh).astype(f32).reshape(1, -1)

    l1t = lin1_W.astype(f32).T
    l1q, l1r = l1t[:_D, :], l1t[_D:, :]
    l1b = lin1_b.astype(f32).reshape(1, -1)
    l2t = lin2_W.astype(f32).T
    l2b = lin2_b.astype(f32).reshape(1, -1)
    batch2d = batch.astype(jnp.int32).reshape(_NN, 1)
    mask1 = jax.random.bernoulli(
        jax.random.key(123), 0.5, (_NG, _D)).astype(f32)
    mask2 = jax.random.bernoulli(
        jax.random.key(456), 0.5, (_NG, 1)).astype(f32)
    dd = jnp.asarray(do_dropout, f32).reshape(1, 1)

    sc_gather, sc_scatter, sc_count = _sc_kernels()

    nblk = 1000
    ngrid = _NN // nblk
    full = lambda shape: pl.BlockSpec(shape, lambda i: (0, 0))
    rows = lambda lanes: pl.BlockSpec((nblk, lanes), lambda i: (i, 0))

    h = pl.pallas_call(
        _lin0_body,
        grid=(ngrid,),
        in_specs=[rows(_F), full((_F, _D)), full((1, _D))],
        out_specs=rows(_D),
        out_shape=jax.ShapeDtypeStruct((_NN, _D), f32),
    )(x, l0t, l0b)

    cnt_part = sc_count(dst2d, ones_ch, zeros_acc)
    c0 = cnt_part[0, :_NN]
    c1 = cnt_part[1, :_NN]

    grid = _NE_PAD // _EBLK
    wf_call = pl.pallas_call(
        _wf_body,
        grid=(grid,),
        in_specs=[
            pl.BlockSpec((_EBLK, 4), lambda i: (i, 0)),
            pl.BlockSpec((4, 128), lambda i: (0, 0)),
            pl.BlockSpec((1, 128), lambda i: (0, 0)),
            pl.BlockSpec((128, 256), lambda i: (0, 0)),
            pl.BlockSpec((1, 256), lambda i: (0, 0)),
        ],
        out_specs=pl.BlockSpec((_EBLK, 256), lambda i: (i, 0)),
        out_shape=jax.ShapeDtypeStruct((_NE_PAD, 256), jnp.bfloat16),
    )
    wf = wf_call(ea_p, w1t, b1, w2t, b2)
    msg_call = pl.pallas_call(
        _msg_body,
        grid=(grid,),
        in_specs=[
            pl.BlockSpec((_EBLK, 256), lambda i: (i, 0)),
            pl.BlockSpec((_EBLK, _D), lambda i: (i, 0)),
            pl.BlockSpec((_D, 256), lambda i: (0, 0)),
        ],
        out_specs=pl.BlockSpec((_EBLK, _D), lambda i: (i, 0)),
        out_shape=jax.ShapeDtypeStruct((_NE_PAD, _D), f32),
    )

    node_call = pl.pallas_call(
        _node_body,
        grid=(ngrid,),
        in_specs=[rows(_D)] * 5 + [full((_D, _D)), full((1, _D))]
        + [full((_D, _D))] * 6 + [full((1, _D))] * 6,
        out_specs=rows(_D),
        out_shape=jax.ShapeDtypeStruct((_NN, _D), f32),
    )

    out = h
    for _ in range(3):
        xj = sc_gather(out, src2d)
        msg = msg_call(wf, xj, rmat)
        s_part = sc_scatter(msg, dst2d, zeros_acc)
        out = node_call(
            s_part[0, :_NN], s_part[1, :_NN], c0, c1, out, root, cb,
            wir, whr, wiz, whz, win, whn, bir, biz, bin_, bhr, bhz, bhn)

    o = _tc_call(
        _s2s_body, jax.ShapeDtypeStruct((_NG, 1), f32),
        out, batch2d, wq, wr_, wh, bg, l1q, l1r, l1b, l2t, l2b,
        mask1, mask2, dd)
    return o.reshape(-1)
